# baseline (speedup 1.0000x reference)
import functools

import jax
import jax.numpy as jnp
from jax import lax
from jax.experimental import pallas as pl
from jax.experimental.pallas import tpu as pltpu
from jax.experimental.pallas import tpu_sc as plsc

NUM_WORDS = 1000000
D = 64
B = 16384
NC, NS, L = 2, 16, 16
NW = NC * NS
BPW = B // NW
CH = 128
NCHUNK = BPW // CH
BROWS = 8192


def _glove_sc(ci_hbm, xi_hbm, wc2_hbm, wx2_hbm, bc2_hbm, bx2_hbm, out_hbm,
              ci_v, xi_v, cit_v, xit_v, bct_v, bxt_v,
              wcr_v, wxr_v, bcr_v, bxr_v, out_v, sem):
    wid = lax.axis_index("s") * NC + lax.axis_index("c")
    base = wid * BPW

    pltpu.sync_copy(ci_hbm.at[pl.ds(base, BPW)], ci_v)
    pltpu.sync_copy(xi_hbm.at[pl.ds(base, BPW)], xi_v)

    def stage(g, carry):
        c = ci_v[pl.ds(g * L, L)]
        x = xi_v[pl.ds(g * L, L)]
        cit_v[pl.ds(g * L, L)] = lax.shift_right_logical(c, 1)
        xit_v[pl.ds(g * L, L)] = lax.shift_right_logical(x, 1)
        bct_v[pl.ds(g * L, L)] = lax.shift_right_logical(c, 7)
        bxt_v[pl.ds(g * L, L)] = lax.shift_right_logical(x, 7)
        return carry

    lax.fori_loop(0, BPW // L, stage, 0)

    lvec = lax.iota(jnp.int32, L)

    def chunk(c, carry):
        off = pl.multiple_of(c * CH, CH)
        c1 = pltpu.async_copy(wc2_hbm.at[cit_v.at[pl.ds(off, CH)]], wcr_v, sem)
        c2 = pltpu.async_copy(wx2_hbm.at[xit_v.at[pl.ds(off, CH)]], wxr_v, sem)
        c3 = pltpu.async_copy(bc2_hbm.at[bct_v.at[pl.ds(off, CH)]], bcr_v, sem)
        c4 = pltpu.async_copy(bx2_hbm.at[bxt_v.at[pl.ds(off, CH)]], bxr_v, sem)
        c1.wait()
        c2.wait()
        c3.wait()
        c4.wait()
        for gi in range(CH // L):
            n_vec = gi * L + lvec
            cw = ci_v[pl.ds(off + gi * L, L)]
            xw = xi_v[pl.ds(off + gi * L, L)]
            col_c = lax.bitwise_and(cw, jnp.full((L,), 1, jnp.int32)) * D
            col_x = lax.bitwise_and(xw, jnp.full((L,), 1, jnp.int32)) * D
            bl_c = lax.bitwise_and(cw, jnp.full((L,), 127, jnp.int32))
            bl_x = lax.bitwise_and(xw, jnp.full((L,), 127, jnp.int32))
            acc = plsc.load_gather(bcr_v, [n_vec, bl_c]) + plsc.load_gather(
                bxr_v, [n_vec, bl_x])
            for d in range(D):
                dcol = jnp.full((L,), d, jnp.int32)
                a = plsc.load_gather(wcr_v, [n_vec, col_c + dcol])
                b = plsc.load_gather(wxr_v, [n_vec, col_x + dcol])
                acc = acc + a * b
            out_v[pl.ds(off + gi * L, L)] = acc
        return carry

    lax.fori_loop(0, NCHUNK, chunk, 0)

    pltpu.sync_copy(out_v, out_hbm.at[pl.ds(base, BPW)])


@jax.jit
def _launch(ci, xi, wc2, wx2, bc2, bx2):
    mesh = plsc.VectorSubcoreMesh(core_axis_name="c", subcore_axis_name="s")
    run = pl.kernel(
        _glove_sc,
        out_type=jax.ShapeDtypeStruct((B,), jnp.float32),
        mesh=mesh,
        scratch_types=[
            pltpu.VMEM((BPW,), jnp.int32),
            pltpu.VMEM((BPW,), jnp.int32),
            pltpu.VMEM((BPW,), jnp.int32),
            pltpu.VMEM((BPW,), jnp.int32),
            pltpu.VMEM((BPW,), jnp.int32),
            pltpu.VMEM((BPW,), jnp.int32),
            pltpu.VMEM((CH, 128), jnp.float32),
            pltpu.VMEM((CH, 128), jnp.float32),
            pltpu.VMEM((CH, 128), jnp.float32),
            pltpu.VMEM((CH, 128), jnp.float32),
            pltpu.VMEM((BPW,), jnp.float32),
            pltpu.SemaphoreType.DMA,
        ],
        compiler_params=pltpu.CompilerParams(needs_layout_passes=False),
    )
    return run(ci, xi, wc2, wx2, bc2, bx2)


def kernel(indices, w_center, w_contex, b_center, b_contex):
    ci = indices[:, 0].astype(jnp.int32)
    xi = indices[:, 1].astype(jnp.int32)
    wc2 = w_center.reshape(NUM_WORDS // 2, 2 * D)
    wx2 = w_contex.reshape(NUM_WORDS // 2, 2 * D)
    pad = BROWS * 128 - NUM_WORDS
    bc2 = jnp.pad(b_center, (0, pad)).reshape(BROWS, 128)
    bx2 = jnp.pad(b_contex, (0, pad)).reshape(BROWS, 128)
    return _launch(ci, xi, wc2, wx2, bc2, bx2)


# speedup vs baseline: 1.5066x; 1.5066x over previous
import functools

import jax
import jax.numpy as jnp
from jax import lax
from jax.experimental import pallas as pl
from jax.experimental.pallas import tpu as pltpu
from jax.experimental.pallas import tpu_sc as plsc

NUM_WORDS = 1000000
D = 64
B = 16384
NC, NS, L = 2, 16, 16
NW = NC * NS
BPW = B // NW
NG = BPW // L
BROWS = 8192


def _glove_sc(ci_hbm, xi_hbm, wc_hbm, wx_hbm, bc2_hbm, bx2_hbm, out_hbm,
              ci_v, xi_v, bct_v, bxt_v,
              wcr_v, wxr_v, bcr_v, bxr_v, out_v, sem):
    wid = lax.axis_index("s") * NC + lax.axis_index("c")
    base = wid * BPW

    pltpu.sync_copy(ci_hbm.at[pl.ds(base, BPW)], ci_v)
    pltpu.sync_copy(xi_hbm.at[pl.ds(base, BPW)], xi_v)

    def stage(g, carry):
        ci_g = ci_v[pl.ds(g * L, L)]
        xi_g = xi_v[pl.ds(g * L, L)]
        bct_v[pl.ds(g * L, L)] = lax.shift_right_logical(ci_g, 7)
        bxt_v[pl.ds(g * L, L)] = lax.shift_right_logical(xi_g, 7)
        return carry

    lax.fori_loop(0, NG, stage, 0)

    lvec = lax.iota(jnp.int32, L)
    zeros = jnp.zeros((L,), jnp.int32)

    def group(g, carry):
        off = pl.multiple_of(g * L, L)
        cw = ci_v[pl.ds(off, L)]
        xw = xi_v[pl.ds(off, L)]
        cb = pltpu.async_copy(bc2_hbm.at[bct_v.at[pl.ds(off, L)]], bcr_v, sem)
        xb = pltpu.async_copy(bx2_hbm.at[bxt_v.at[pl.ds(off, L)]], bxr_v, sem)
        copies = []
        for k in range(L):
            kf = jnp.full((L,), k, jnp.int32)
            sc = lax.reduce_sum(jnp.where(lvec == kf, cw, zeros), axes=(0,))
            sx = lax.reduce_sum(jnp.where(lvec == kf, xw, zeros), axes=(0,))
            copies.append(pltpu.async_copy(
                wc_hbm.at[pl.ds(sc, 1), :],
                wcr_v.at[pl.ds(k, 1), :], sem))
            copies.append(pltpu.async_copy(
                wx_hbm.at[pl.ds(sx, 1), :],
                wxr_v.at[pl.ds(k, 1), :], sem))
        for c in copies:
            c.wait()
        cb.wait()
        xb.wait()
        bl_c = lax.bitwise_and(cw, jnp.full((L,), 127, jnp.int32))
        bl_x = lax.bitwise_and(xw, jnp.full((L,), 127, jnp.int32))
        acc = plsc.load_gather(bcr_v, [lvec, bl_c]) + plsc.load_gather(
            bxr_v, [lvec, bl_x])
        for d in range(D):
            dcol = jnp.full((L,), d, jnp.int32)
            a = plsc.load_gather(wcr_v, [lvec, dcol])
            b = plsc.load_gather(wxr_v, [lvec, dcol])
            acc = acc + a * b
        out_v[pl.ds(off, L)] = acc
        return carry

    lax.fori_loop(0, NG, group, 0)

    pltpu.sync_copy(out_v, out_hbm.at[pl.ds(base, BPW)])


@jax.jit
def _launch(ci, xi, wc, wx, bc2, bx2):
    mesh = plsc.VectorSubcoreMesh(core_axis_name="c", subcore_axis_name="s")
    run = pl.kernel(
        _glove_sc,
        out_type=jax.ShapeDtypeStruct((B,), jnp.float32),
        mesh=mesh,
        scratch_types=[
            pltpu.VMEM((BPW,), jnp.int32),
            pltpu.VMEM((BPW,), jnp.int32),
            pltpu.VMEM((BPW,), jnp.int32),
            pltpu.VMEM((BPW,), jnp.int32),
            pltpu.VMEM((L, D), jnp.float32),
            pltpu.VMEM((L, D), jnp.float32),
            pltpu.VMEM((L, 128), jnp.float32),
            pltpu.VMEM((L, 128), jnp.float32),
            pltpu.VMEM((BPW,), jnp.float32),
            pltpu.SemaphoreType.DMA,
        ],
        compiler_params=pltpu.CompilerParams(needs_layout_passes=False),
    )
    return run(ci, xi, wc, wx, bc2, bx2)


def kernel(indices, w_center, w_contex, b_center, b_contex):
    ci = indices[:, 0].astype(jnp.int32)
    xi = indices[:, 1].astype(jnp.int32)
    pad = BROWS * 128 - NUM_WORDS
    bc2 = jnp.pad(b_center, (0, pad)).reshape(BROWS, 128)
    bx2 = jnp.pad(b_contex, (0, pad)).reshape(BROWS, 128)
    return _launch(ci, xi, w_center, w_contex, bc2, bx2)


# speedup vs baseline: 1.5345x; 1.0185x over previous
import functools

import jax
import jax.numpy as jnp
from jax import lax
from jax.experimental import pallas as pl
from jax.experimental.pallas import tpu as pltpu
from jax.experimental.pallas import tpu_sc as plsc

NUM_WORDS = 1000000
D = 64
B = 16384
NC, NS, L = 2, 16, 16
NW = NC * NS
BPW = B // NW
NG = BPW // L
CH = 64
BROWS = 8192


def _glove_sc(ci_hbm, xi_hbm, wc_hbm, wx_hbm, bc2_hbm, bx2_hbm, out_hbm,
              ci_v, xi_v, bct_v, bxt_v,
              wcr_v, wxr_v, bcr_v, bxr_v, out_v, sem):
    wid = lax.axis_index("s") * NC + lax.axis_index("c")
    base = wid * BPW

    pltpu.sync_copy(ci_hbm.at[pl.ds(base, BPW)], ci_v)
    pltpu.sync_copy(xi_hbm.at[pl.ds(base, BPW)], xi_v)

    def stage(g, carry):
        ci_g = ci_v[pl.ds(g * L, L)]
        xi_g = xi_v[pl.ds(g * L, L)]
        bct_v[pl.ds(g * L, L)] = lax.shift_right_logical(ci_g, 7)
        bxt_v[pl.ds(g * L, L)] = lax.shift_right_logical(xi_g, 7)
        return carry

    lax.fori_loop(0, NG, stage, 0)

    lvec = lax.iota(jnp.int32, L)
    zeros = jnp.zeros((L,), jnp.int32)

    def chunk(c, carry):
        off = pl.multiple_of(c * CH, CH)
        cb = pltpu.async_copy(bc2_hbm.at[bct_v.at[pl.ds(off, CH)]], bcr_v, sem)
        xb = pltpu.async_copy(bx2_hbm.at[bxt_v.at[pl.ds(off, CH)]], bxr_v, sem)
        copies = []
        for gi in range(CH // L):
            cw = ci_v[pl.ds(off + gi * L, L)]
            xw = xi_v[pl.ds(off + gi * L, L)]
            for k in range(L):
                kf = jnp.full((L,), k, jnp.int32)
                sc = lax.reduce_sum(jnp.where(lvec == kf, cw, zeros), axes=(0,))
                sx = lax.reduce_sum(jnp.where(lvec == kf, xw, zeros), axes=(0,))
                copies.append(pltpu.async_copy(
                    wc_hbm.at[pl.ds(sc, 1), :],
                    wcr_v.at[pl.ds(gi * L + k, 1), :], sem))
                copies.append(pltpu.async_copy(
                    wx_hbm.at[pl.ds(sx, 1), :],
                    wxr_v.at[pl.ds(gi * L + k, 1), :], sem))
        for cp in copies:
            cp.wait()
        cb.wait()
        xb.wait()
        for gi in range(CH // L):
            n_vec = gi * L + lvec
            cw = ci_v[pl.ds(off + gi * L, L)]
            xw = xi_v[pl.ds(off + gi * L, L)]
            bl_c = lax.bitwise_and(cw, jnp.full((L,), 127, jnp.int32))
            bl_x = lax.bitwise_and(xw, jnp.full((L,), 127, jnp.int32))
            acc = plsc.load_gather(bcr_v, [n_vec, bl_c]) + plsc.load_gather(
                bxr_v, [n_vec, bl_x])
            for d in range(D):
                dcol = jnp.full((L,), d, jnp.int32)
                a = plsc.load_gather(wcr_v, [n_vec, dcol])
                b = plsc.load_gather(wxr_v, [n_vec, dcol])
                acc = acc + a * b
            out_v[pl.ds(off + gi * L, L)] = acc
        return carry

    lax.fori_loop(0, BPW // CH, chunk, 0)

    pltpu.sync_copy(out_v, out_hbm.at[pl.ds(base, BPW)])


@jax.jit
def _launch(ci, xi, wc, wx, bc2, bx2):
    mesh = plsc.VectorSubcoreMesh(core_axis_name="c", subcore_axis_name="s")
    run = pl.kernel(
        _glove_sc,
        out_type=jax.ShapeDtypeStruct((B,), jnp.float32),
        mesh=mesh,
        scratch_types=[
            pltpu.VMEM((BPW,), jnp.int32),
            pltpu.VMEM((BPW,), jnp.int32),
            pltpu.VMEM((BPW,), jnp.int32),
            pltpu.VMEM((BPW,), jnp.int32),
            pltpu.VMEM((CH, D), jnp.float32),
            pltpu.VMEM((CH, D), jnp.float32),
            pltpu.VMEM((CH, 128), jnp.float32),
            pltpu.VMEM((CH, 128), jnp.float32),
            pltpu.VMEM((BPW,), jnp.float32),
            pltpu.SemaphoreType.DMA,
        ],
        compiler_params=pltpu.CompilerParams(needs_layout_passes=False),
    )
    return run(ci, xi, wc, wx, bc2, bx2)


def kernel(indices, w_center, w_contex, b_center, b_contex):
    ci = indices[:, 0].astype(jnp.int32)
    xi = indices[:, 1].astype(jnp.int32)
    pad = BROWS * 128 - NUM_WORDS
    bc2 = jnp.pad(b_center, (0, pad)).reshape(BROWS, 128)
    bx2 = jnp.pad(b_contex, (0, pad)).reshape(BROWS, 128)
    return _launch(ci, xi, w_center, w_contex, bc2, bx2)
